# Initial kernel scaffold; baseline (speedup 1.0000x reference)
import jax
import jax.numpy as jnp
from jax.experimental import pallas as pl


def kernel(x, edge_index, edge_attr, batch_vec, W1, W2, b2, Wc1, bc1, Wc2, bc2, Wl, bl):
    raise NotImplementedError("write your pallas kernel here")


# speedup vs baseline: 17.9901x; 17.9901x over previous
import functools

import jax
import jax.numpy as jnp
from jax import lax
from jax.experimental import pallas as pl
from jax.experimental.pallas import tpu as pltpu
from jax.experimental.pallas import tpu_sc as plsc

N = 32768
E = 524288
BS = 256
F_IN = 128
HID = 512
CNN = 256
E2 = E + N
NC, NS = 2, 16
NW = NC * NS
FCH = 16
ER = E2 // 128
ER_W = ER // NW
ER_S = ER // NS
BROWS = 8
NB = ER_S // BROWS

_mesh = lambda: plsc.VectorSubcoreMesh(
    core_axis_name="c", subcore_axis_name="s", num_cores=NC, num_subcores=NS)


def _deg_body(colm, ewm, out, colbuf, ewbuf, degpart):
    c = lax.axis_index("c")
    s = lax.axis_index("s")
    w = s * NC + c
    zf = jnp.zeros((16,), jnp.float32)

    def zero(i, _):
        degpart[pl.ds(i * 16, 16)] = zf
        return 0
    lax.fori_loop(0, N // 16, zero, 0)

    base = w * ER_W
    pltpu.sync_copy(colm.at[pl.ds(base, ER_W)], colbuf)
    pltpu.sync_copy(ewm.at[pl.ds(base, ER_W)], ewbuf)

    def body(r, _):
        for k in range(8):
            idx = colbuf[r, pl.ds(k * 16, 16)]
            wv = ewbuf[r, pl.ds(k * 16, 16)]
            plsc.addupdate_scatter(degpart, [idx], wv)
        return 0
    lax.fori_loop(0, ER_W, body, 0)
    pltpu.sync_copy(degpart, out.at[w])


def _deg_call(colm, ewm):
    return pl.kernel(
        _deg_body,
        out_type=jax.ShapeDtypeStruct((NW, N), jnp.float32),
        mesh=_mesh(),
        scratch_types=[
            pltpu.VMEM((ER_W, 128), jnp.int32),
            pltpu.VMEM((ER_W, 128), jnp.float32),
            pltpu.VMEM((N,), jnp.float32),
        ],
        compiler_params=pltpu.CompilerParams(needs_layout_passes=False),
    )(colm, ewm)


def _dinv_body(parts_ref, out_ref):
    sdeg = jnp.sum(parts_ref[...], axis=0)
    out_ref[...] = jnp.where(sdeg > 0, lax.rsqrt(sdeg), 0.0).reshape(256, 128)


def _dinv_call(parts):
    return pl.pallas_call(
        _dinv_body,
        out_shape=jax.ShapeDtypeStruct((256, 128), jnp.float32),
    )(parts)


def _prop_body(x4, rowm, colm, ewm, dinv_h, out_h,
               dinvbuf, rowbuf, colbuf, ewbuf, gidx, normbuf, rowsbuf, zbuf,
               accum, sem):
    c = lax.axis_index("c")
    s = lax.axis_index("s")
    pltpu.sync_copy(dinv_h, dinvbuf)
    zf = jnp.zeros((16,), jnp.float32)

    def zero(i, _):
        zbuf[i, pl.ds(0, 16)] = zf
        return 0
    lax.fori_loop(0, 1024, zero, 0)

    for cl in range(4):
        chunk = c * 4 + cl
        coff = chunk * N
        pltpu.sync_copy(zbuf, accum.at[pl.ds(s * 2048, 1024)])
        pltpu.sync_copy(zbuf, accum.at[pl.ds(s * 2048 + 1024, 1024)])
        plsc.subcore_barrier()

        def batch(b, _):
            rbase = s * ER_S + b * BROWS
            pltpu.sync_copy(rowm.at[pl.ds(rbase, BROWS)], rowbuf)
            pltpu.sync_copy(colm.at[pl.ds(rbase, BROWS)], colbuf)
            pltpu.sync_copy(ewm.at[pl.ds(rbase, BROWS)], ewbuf)
            for k in range(BROWS):
                def norms(j, _):
                    r16 = rowbuf[k, pl.ds(j * 16, 16)]
                    c16 = colbuf[k, pl.ds(j * 16, 16)]
                    w16 = ewbuf[k, pl.ds(j * 16, 16)]
                    dr = plsc.load_gather(dinvbuf, [r16])
                    dc = plsc.load_gather(dinvbuf, [c16])
                    normbuf[k, pl.ds(j * 16, 16)] = dr * w16 * dc
                    gidx[k, pl.ds(j * 16, 16)] = r16 + coff
                    return 0
                lax.fori_loop(0, 8, norms, 0)
            cps = [pltpu.make_async_copy(
                x4.at[gidx.at[k]], rowsbuf.at[pl.ds(k * 128, 128)], sem)
                for k in range(BROWS)]
            for cp in cps:
                cp.start()
            for cp in cps:
                cp.wait()
            for k in range(BROWS):
                def scale(j, _):
                    nv16 = normbuf[k, pl.ds(j * 16, 16)]
                    for u in range(16):
                        nv = nv16[u]
                        row = k * 128 + j * 16 + u
                        rowsbuf[row, pl.ds(0, 16)] = rowsbuf[row, pl.ds(0, 16)] * nv
                    return 0
                lax.fori_loop(0, 8, scale, 0)
                pltpu.sync_copy(rowsbuf.at[pl.ds(k * 128, 128)],
                                accum.at[colbuf.at[k]], add=True)
            return 0
        lax.fori_loop(0, NB, batch, 0)
        plsc.subcore_barrier()
        pltpu.sync_copy(accum.at[pl.ds(s * 2048, 2048)],
                        out_h.at[pl.ds(coff + s * 2048, 2048)])
        plsc.subcore_barrier()


def _prop_call(x4, rowm, colm, ewm, dinv):
    return pl.kernel(
        _prop_body,
        out_type=jax.ShapeDtypeStruct((8 * N, FCH), jnp.float32),
        mesh=_mesh(),
        scratch_types=[
            pltpu.VMEM((N,), jnp.float32),
            pltpu.VMEM((BROWS, 128), jnp.int32),
            pltpu.VMEM((BROWS, 128), jnp.int32),
            pltpu.VMEM((BROWS, 128), jnp.float32),
            pltpu.VMEM((BROWS, 128), jnp.int32),
            pltpu.VMEM((BROWS, 128), jnp.float32),
            pltpu.VMEM((1024, FCH), jnp.float32),
            pltpu.VMEM((1024, FCH), jnp.float32),
            pltpu.VMEM_SHARED((N, FCH), jnp.float32),
            pltpu.SemaphoreType.DMA,
        ],
        compiler_params=pltpu.CompilerParams(
            needs_layout_passes=False, use_tc_tiling_on_sc=False),
    )(x4, rowm, colm, ewm, dinv)


def _head_body(p_ref, w1, w2, b2r, wc1, bc1r, wc2, bc2r, wl2r, blr, o_ref):
    p = p_ref[...]
    w12 = jnp.dot(w1[...], w2[...], preferred_element_type=jnp.float32)
    g = jnp.tanh(jnp.dot(p, w12, preferred_element_type=jnp.float32)
                 + b2r[...])
    g4 = g.reshape(16, 4, 32, HID)
    acc = jnp.broadcast_to(bc1r[...], (512, HID)).astype(jnp.float32)
    for gi in range(4):
        ggi = g4[:, gi].reshape(512, HID)
        wslice = wc1[...][:, gi * HID:(gi + 1) * HID]
        acc = acc + lax.dot_general(
            ggi, wslice, (((1,), (1,)), ((), ())),
            preferred_element_type=jnp.float32)
    a = jax.nn.relu(acc)
    bm = jax.nn.relu(lax.dot_general(
        a, wc2[...], (((1,), (1,)), ((), ())),
        preferred_element_type=jnp.float32) + bc2r[...])
    bm3 = bm.reshape(16, 32, CNN)
    pr = bm3 * wl2r[...][None, :, :]
    sv = jnp.sum(jnp.sum(pr, axis=2), axis=1) + blr[0, 0]
    o_ref[...] = jax.nn.sigmoid(sv).reshape(1, 1, 16)


def _head_call(p, W1, W2, b2, Wc1, bc1, Wc2, bc2, wl2, bl):
    full = lambda shape: pl.BlockSpec(shape, lambda i: tuple(0 for _ in shape))
    return pl.pallas_call(
        _head_body,
        grid=(16,),
        in_specs=[
            pl.BlockSpec((2048, F_IN), lambda i: (i, 0)),
            full((F_IN, F_IN)), full((F_IN, HID)), full((1, HID)),
            full((HID, 4 * HID)), full((1, HID)),
            full((CNN, HID)), full((1, CNN)),
            full((32, CNN)), full((1, 1)),
        ],
        out_specs=pl.BlockSpec((1, 1, 16), lambda i: (i, 0, 0)),
        out_shape=jax.ShapeDtypeStruct((16, 1, 16), jnp.float32),
    )(p, W1, W2, b2, Wc1, bc1, Wc2, bc2, wl2, bl)


def kernel(x, edge_index, edge_attr, batch_vec, W1, W2, b2, Wc1, bc1, Wc2,
           bc2, Wl, bl):
    loop = jnp.arange(N, dtype=jnp.int32)
    rowm = jnp.concatenate([edge_index[0], loop]).reshape(ER, 128)
    colm = jnp.concatenate([edge_index[1], loop]).reshape(ER, 128)
    ewm = jnp.concatenate(
        [edge_attr, jnp.ones((N,), jnp.float32)]).reshape(ER, 128)
    x4 = x.reshape(N, 8, FCH).transpose(1, 0, 2).reshape(8 * N, FCH)

    parts = _deg_call(colm, ewm)
    dinv = _dinv_call(parts).reshape(N)
    prop4 = _prop_call(x4, rowm, colm, ewm, dinv)
    prop = prop4.reshape(8, N, FCH).transpose(1, 0, 2).reshape(N, F_IN)
    out = _head_call(prop, W1, W2, b2.reshape(1, HID), Wc1,
                     bc1.reshape(1, HID), Wc2, bc2.reshape(1, CNN),
                     Wl.reshape(CNN, 32).T, bl.reshape(1, 1))
    return out.reshape(-1)
